# baseline (speedup 1.0000x reference)
import numpy as np

import jax
import jax.numpy as jnp
from jax.experimental import pallas as pl
from jax.experimental.pallas import tpu as pltpu

_TB = 128

_XSEQ = list(range(0, 32, 2)) + list(range(1, 32, 2))
_TAPS = {0: [(0, 15), (16, 31)],
         1: [(16, 31), (1, 16)],
         2: [(1, 16), (17, 32)]}


def _fused_kernel(x_ref,
                  m1_ref,
                  b1_ref,
                  m2_ref,
                  b2_ref,
                  fw1_ref, fb1_ref,
                  fw2_ref, fb2_ref,
                  fw3_ref, fb3_ref,
                  o_ref):
    f32 = jnp.float32
    vt3 = jnp.transpose(x_ref[...]).reshape(32, 32, _TB)

    xp = jnp.concatenate([vt3[:, c, :] for c in _XSEQ], axis=1)

    a1s = jnp.concatenate(
        [jnp.concatenate([xp[:, lo * _TB:hi * _TB] for lo, hi in _TAPS[j]],
                         axis=1)
         for j in range(3)], axis=0)
    c1 = jnp.dot(m1_ref[...], a1s, preferred_element_type=f32)

    px = jnp.maximum(c1[:, 0:1920], c1[:, 1920:3840])
    a2 = jnp.maximum(jnp.maximum(px[0:96, :], px[96:192, :])
                     + b1_ref[...], 0.0)

    c2 = (jnp.dot(m2_ref[0, :, :], a2[:, 0:1664],
                  preferred_element_type=f32)
          + jnp.dot(m2_ref[1, :, :], a2[:, 128:1792],
                    preferred_element_type=f32)
          + jnp.dot(m2_ref[2, :, :], a2[:, 256:1920],
                    preferred_element_type=f32))

    p2y = jnp.maximum(c2[0:96, :], c2[96:192, :])
    hf = jnp.concatenate(
        [jnp.maximum(p2y[:, 256 * k:256 * k + _TB],
                     p2y[:, 256 * k + _TB:256 * k + 2 * _TB])
         for k in range(6)], axis=0)
    hf = jnp.maximum(hf + b2_ref[...], 0.0)

    h3 = jnp.maximum(jnp.dot(fw1_ref[...], hf,
                             preferred_element_type=f32) + fb1_ref[...], 0.0)
    h4 = jnp.maximum(jnp.dot(fw2_ref[...], h3,
                             preferred_element_type=f32) + fb2_ref[...], 0.0)
    z = jnp.dot(fw3_ref[...], h4,
                preferred_element_type=f32) + fb3_ref[...]
    o_ref[...] = 1.0 / (1.0 + jnp.exp(-z))


def _band(n_out, n_in, s, i):
    b = np.zeros((n_out, n_in), np.float32)
    b[np.arange(n_out), 2 * np.arange(n_out) + s + i] = 1.0
    return b


def _build_conv1_matrix(w1):
    w = w1[:, 0].astype(jnp.float32)
    sblocks = []
    for s in range(2):
        jcols = []
        for j in range(3):
            mj = sum(jnp.asarray(_band(15, 32, s, i))[:, None, :]
                     * w[None, :, i, j, None] for i in range(3))
            jcols.append(mj.reshape(90, 32))
        blk = jnp.concatenate(jcols, axis=1)
        sblocks.append(jnp.pad(blk, ((0, 6), (0, 0))))
    return jnp.concatenate(sblocks, axis=0)


def _build_conv2_matrix(w2):
    w = w2.astype(jnp.float32)
    mats = []
    for j in range(3):
        sblocks = []
        for s in range(2):
            mj = sum(jnp.asarray(_band(6, 15, s, i))[:, None, :, None]
                     * w[None, :, None, :, i, j] for i in range(3))
            sblocks.append(jnp.pad(mj.reshape(96, 90), ((0, 0), (0, 6))))
        mats.append(jnp.concatenate(sblocks, axis=0))
    return jnp.stack(mats, axis=0)


def kernel(conv1_w, conv1_b, conv2_w, conv2_b, fc1_w, fc1_b,
           fc2_w, fc2_b, fc3_w, fc3_b, x_nchw):
    f32 = jnp.float32
    n = x_nchw.shape[0]
    n_pad = ((n + _TB - 1) // _TB) * _TB
    t = n_pad // _TB

    x = jnp.asarray(x_nchw, f32).reshape(n, 32 * 32)
    xa = jnp.pad(x, ((0, n_pad - n), (0, 0)))

    m1 = _build_conv1_matrix(conv1_w)
    b1c = jnp.pad(jnp.tile(conv1_b.astype(f32), (15,)), (0, 6)).reshape(96, 1)
    m2 = _build_conv2_matrix(conv2_w)
    b2c = jnp.tile(conv2_b.astype(f32), (36,)).reshape(576, 1)
    fw1 = fc1_w.reshape(128, 16, 6, 6).transpose(0, 3, 2, 1).reshape(128, 576)
    fw1 = fw1.astype(f32)
    fb1 = fc1_b.reshape(128, 1).astype(f32)
    fw2 = fc2_w.astype(f32)
    fb2 = fc2_b.reshape(64, 1).astype(f32)
    fw3 = fc3_w.astype(f32)
    fb3 = fc3_b.reshape(1, 1).astype(f32)

    out = pl.pallas_call(
        _fused_kernel,
        out_shape=jax.ShapeDtypeStruct((1, n_pad), f32),
        grid=(t,),
        in_specs=[
            pl.BlockSpec((_TB, 1024), lambda i: (i, 0)),
            pl.BlockSpec((192, 96), lambda i: (0, 0)),
            pl.BlockSpec((96, 1), lambda i: (0, 0)),
            pl.BlockSpec((3, 192, 96), lambda i: (0, 0, 0)),
            pl.BlockSpec((576, 1), lambda i: (0, 0)),
            pl.BlockSpec((128, 576), lambda i: (0, 0)),
            pl.BlockSpec((128, 1), lambda i: (0, 0)),
            pl.BlockSpec((64, 128), lambda i: (0, 0)),
            pl.BlockSpec((64, 1), lambda i: (0, 0)),
            pl.BlockSpec((1, 64), lambda i: (0, 0)),
            pl.BlockSpec((1, 1), lambda i: (0, 0)),
        ],
        out_specs=pl.BlockSpec((1, _TB), lambda i: (0, i)),
        compiler_params=pltpu.CompilerParams(
            dimension_semantics=("parallel",)),
    )(xa, m1, b1c, m2, b2c, fw1, fb1, fw2, fb2, fw3, fb3)

    return jnp.transpose(out[:, :n])


# speedup vs baseline: 1.0397x; 1.0397x over previous
import numpy as np

import jax
import jax.numpy as jnp
from jax.experimental import pallas as pl
from jax.experimental.pallas import tpu as pltpu

_TB = 128

_XSEQ = list(range(0, 32, 2)) + list(range(1, 32, 2))
_TAPS = {0: [(0, 15), (16, 31)],
         1: [(16, 31), (1, 16)],
         2: [(1, 16), (17, 32)]}


def _fused_kernel(x_ref,
                  m1_ref,
                  b1_ref,
                  m2_ref,
                  b2_ref,
                  fw1_ref, fb1_ref,
                  fw2_ref, fb2_ref,
                  fw3_ref, fb3_ref,
                  o_ref):
    f32 = jnp.float32
    vt3 = jnp.transpose(x_ref[...]).reshape(32, 32, _TB)

    xp = jnp.concatenate([vt3[:, c, :] for c in _XSEQ], axis=1)

    a1s = jnp.concatenate(
        [jnp.concatenate([xp[:, lo * _TB:hi * _TB] for lo, hi in _TAPS[j]],
                         axis=1)
         for j in range(3)], axis=0)
    c1 = jnp.dot(m1_ref[...], a1s, preferred_element_type=f32)

    px = jnp.maximum(c1[:, 0:1920], c1[:, 1920:3840])
    a2 = jnp.maximum(jnp.maximum(px[0:96, :], px[96:192, :])
                     + b1_ref[...], 0.0)

    c2 = (jnp.dot(m2_ref[0, :, :], a2[:, 0:1664],
                  preferred_element_type=f32)
          + jnp.dot(m2_ref[1, :, :], a2[:, 128:1792],
                    preferred_element_type=f32)
          + jnp.dot(m2_ref[2, :, :], a2[:, 256:1920],
                    preferred_element_type=f32))

    p2y = jnp.maximum(c2[0:96, :], c2[96:192, :])
    hf = jnp.concatenate(
        [jnp.maximum(p2y[:, 256 * k:256 * k + _TB],
                     p2y[:, 256 * k + _TB:256 * k + 2 * _TB])
         for k in range(6)], axis=0)
    hf = jnp.maximum(hf + b2_ref[...], 0.0)

    h3 = jnp.maximum(jnp.dot(fw1_ref[...], hf,
                             preferred_element_type=f32) + fb1_ref[...], 0.0)
    h4 = jnp.maximum(jnp.dot(fw2_ref[...], h3,
                             preferred_element_type=f32) + fb2_ref[...], 0.0)
    z = jnp.dot(fw3_ref[...], h4,
                preferred_element_type=f32) + fb3_ref[...]
    o_ref[...] = 1.0 / (1.0 + jnp.exp(-z))


def _band_tensor(n_out, n_in):
    b = np.zeros((2, 3, n_out, n_in), np.float32)
    for s in range(2):
        for i in range(3):
            b[s, i, np.arange(n_out), 2 * np.arange(n_out) + s + i] = 1.0
    return b


_B1 = _band_tensor(15, 32)
_B2 = _band_tensor(6, 15)


def _build_conv1_matrix(w1):
    w = w1[:, 0].astype(jnp.float32)
    m = jnp.einsum("cij,siyu->sycju", w, jnp.asarray(_B1))
    return jnp.pad(m.reshape(2, 90, 96), ((0, 0), (0, 6), (0, 0))).reshape(192, 96)


def _build_conv2_matrix(w2):
    w = w2.astype(jnp.float32)
    m = jnp.einsum("ocij,siyk->jsyokc", w, jnp.asarray(_B2))
    return jnp.pad(m.reshape(3, 192, 90), ((0, 0), (0, 0), (0, 6)))


def kernel(conv1_w, conv1_b, conv2_w, conv2_b, fc1_w, fc1_b,
           fc2_w, fc2_b, fc3_w, fc3_b, x_nchw):
    f32 = jnp.float32
    n = x_nchw.shape[0]
    n_pad = ((n + _TB - 1) // _TB) * _TB
    t = n_pad // _TB

    x = jnp.asarray(x_nchw, f32).reshape(n, 32 * 32)
    xa = jnp.pad(x, ((0, n_pad - n), (0, 0)))

    m1 = _build_conv1_matrix(conv1_w)
    b1c = jnp.pad(jnp.tile(conv1_b.astype(f32), (15,)), (0, 6)).reshape(96, 1)
    m2 = _build_conv2_matrix(conv2_w)
    b2c = jnp.tile(conv2_b.astype(f32), (36,)).reshape(576, 1)
    fw1 = fc1_w.reshape(128, 16, 6, 6).transpose(0, 3, 2, 1).reshape(128, 576)
    fw1 = fw1.astype(f32)
    fb1 = fc1_b.reshape(128, 1).astype(f32)
    fw2 = fc2_w.astype(f32)
    fb2 = fc2_b.reshape(64, 1).astype(f32)
    fw3 = fc3_w.astype(f32)
    fb3 = fc3_b.reshape(1, 1).astype(f32)

    out = pl.pallas_call(
        _fused_kernel,
        out_shape=jax.ShapeDtypeStruct((1, n_pad), f32),
        grid=(t,),
        in_specs=[
            pl.BlockSpec((_TB, 1024), lambda i: (i, 0)),
            pl.BlockSpec((192, 96), lambda i: (0, 0)),
            pl.BlockSpec((96, 1), lambda i: (0, 0)),
            pl.BlockSpec((3, 192, 96), lambda i: (0, 0, 0)),
            pl.BlockSpec((576, 1), lambda i: (0, 0)),
            pl.BlockSpec((128, 576), lambda i: (0, 0)),
            pl.BlockSpec((128, 1), lambda i: (0, 0)),
            pl.BlockSpec((64, 128), lambda i: (0, 0)),
            pl.BlockSpec((64, 1), lambda i: (0, 0)),
            pl.BlockSpec((1, 64), lambda i: (0, 0)),
            pl.BlockSpec((1, 1), lambda i: (0, 0)),
        ],
        out_specs=pl.BlockSpec((1, _TB), lambda i: (0, i)),
        compiler_params=pltpu.CompilerParams(
            dimension_semantics=("parallel",)),
    )(xa, m1, b1c, m2, b2c, fw1, fb1, fw2, fb2, fw3, fb3)

    return jnp.transpose(out[:, :n])


# speedup vs baseline: 1.2030x; 1.1570x over previous
import numpy as np

import jax
import jax.numpy as jnp
from jax.experimental import pallas as pl
from jax.experimental.pallas import tpu as pltpu

_TB = 256

_XSEQ = list(range(0, 32, 2)) + list(range(1, 32, 2))
_TAPS = {0: [(0, 15), (16, 31)],
         1: [(16, 31), (1, 16)],
         2: [(1, 16), (17, 32)]}


def _fused_kernel(x_ref,
                  m1_ref,
                  b1_ref,
                  m2_ref,
                  b2_ref,
                  fw1_ref, fb1_ref,
                  fw2_ref, fb2_ref,
                  fw3_ref, fb3_ref,
                  o_ref):
    f32 = jnp.float32
    vt3 = jnp.transpose(x_ref[...]).reshape(32, 32, _TB)

    xp = jnp.concatenate([vt3[:, c, :] for c in _XSEQ], axis=1)

    a1s = jnp.concatenate(
        [jnp.concatenate([xp[:, lo * _TB:hi * _TB] for lo, hi in _TAPS[j]],
                         axis=1)
         for j in range(3)], axis=0)
    c1 = jnp.dot(m1_ref[...], a1s, preferred_element_type=f32)

    px = jnp.maximum(c1[:, 0:15 * _TB], c1[:, 15 * _TB:30 * _TB])
    a2 = jnp.maximum(jnp.maximum(px[0:96, :], px[96:192, :])
                     + b1_ref[...], 0.0)

    c2 = (jnp.dot(m2_ref[0, :, :], a2[:, 0:13 * _TB],
                  preferred_element_type=f32)
          + jnp.dot(m2_ref[1, :, :], a2[:, _TB:14 * _TB],
                    preferred_element_type=f32)
          + jnp.dot(m2_ref[2, :, :], a2[:, 2 * _TB:15 * _TB],
                    preferred_element_type=f32))

    p2y = jnp.maximum(c2[0:96, :], c2[96:192, :])
    hf = jnp.concatenate(
        [jnp.maximum(p2y[:, 2 * k * _TB:(2 * k + 1) * _TB],
                     p2y[:, (2 * k + 1) * _TB:(2 * k + 2) * _TB])
         for k in range(6)], axis=0)
    hf = jnp.maximum(hf + b2_ref[...], 0.0)

    h3 = jnp.maximum(jnp.dot(fw1_ref[...], hf,
                             preferred_element_type=f32) + fb1_ref[...], 0.0)
    h4 = jnp.maximum(jnp.dot(fw2_ref[...], h3,
                             preferred_element_type=f32) + fb2_ref[...], 0.0)
    z = jnp.dot(fw3_ref[...], h4,
                preferred_element_type=f32) + fb3_ref[...]
    o_ref[...] = 1.0 / (1.0 + jnp.exp(-z))


def _band_tensor(n_out, n_in):
    b = np.zeros((2, 3, n_out, n_in), np.float32)
    for s in range(2):
        for i in range(3):
            b[s, i, np.arange(n_out), 2 * np.arange(n_out) + s + i] = 1.0
    return b


_B1 = _band_tensor(15, 32)
_B2 = _band_tensor(6, 15)


def _build_conv1_matrix(w1):
    w = w1[:, 0].astype(jnp.float32)
    m = jnp.einsum("cij,siyu->sycju", w, jnp.asarray(_B1))
    return jnp.pad(m.reshape(2, 90, 96), ((0, 0), (0, 6), (0, 0))).reshape(192, 96)


def _build_conv2_matrix(w2):
    w = w2.astype(jnp.float32)
    m = jnp.einsum("ocij,siyk->jsyokc", w, jnp.asarray(_B2))
    return jnp.pad(m.reshape(3, 192, 90), ((0, 0), (0, 0), (0, 6)))


def kernel(conv1_w, conv1_b, conv2_w, conv2_b, fc1_w, fc1_b,
           fc2_w, fc2_b, fc3_w, fc3_b, x_nchw):
    f32 = jnp.float32
    n = x_nchw.shape[0]
    n_pad = ((n + _TB - 1) // _TB) * _TB
    t = n_pad // _TB

    x = jnp.asarray(x_nchw, f32).reshape(n, 32 * 32)
    xa = jnp.pad(x, ((0, n_pad - n), (0, 0)))

    m1 = _build_conv1_matrix(conv1_w)
    b1c = jnp.pad(jnp.tile(conv1_b.astype(f32), (15,)), (0, 6)).reshape(96, 1)
    m2 = _build_conv2_matrix(conv2_w)
    b2c = jnp.tile(conv2_b.astype(f32), (36,)).reshape(576, 1)
    fw1 = fc1_w.reshape(128, 16, 6, 6).transpose(0, 3, 2, 1).reshape(128, 576)
    fw1 = fw1.astype(f32)
    fb1 = fc1_b.reshape(128, 1).astype(f32)
    fw2 = fc2_w.astype(f32)
    fb2 = fc2_b.reshape(64, 1).astype(f32)
    fw3 = fc3_w.astype(f32)
    fb3 = fc3_b.reshape(1, 1).astype(f32)

    out = pl.pallas_call(
        _fused_kernel,
        out_shape=jax.ShapeDtypeStruct((1, n_pad), f32),
        grid=(t,),
        in_specs=[
            pl.BlockSpec((_TB, 1024), lambda i: (i, 0)),
            pl.BlockSpec((192, 96), lambda i: (0, 0)),
            pl.BlockSpec((96, 1), lambda i: (0, 0)),
            pl.BlockSpec((3, 192, 96), lambda i: (0, 0, 0)),
            pl.BlockSpec((576, 1), lambda i: (0, 0)),
            pl.BlockSpec((128, 576), lambda i: (0, 0)),
            pl.BlockSpec((128, 1), lambda i: (0, 0)),
            pl.BlockSpec((64, 128), lambda i: (0, 0)),
            pl.BlockSpec((64, 1), lambda i: (0, 0)),
            pl.BlockSpec((1, 64), lambda i: (0, 0)),
            pl.BlockSpec((1, 1), lambda i: (0, 0)),
        ],
        out_specs=pl.BlockSpec((1, _TB), lambda i: (0, i)),
        compiler_params=pltpu.CompilerParams(
            dimension_semantics=("parallel",)),
    )(xa, m1, b1c, m2, b2c, fw1, fb1, fw2, fb2, fw3, fb3)

    return jnp.transpose(out[:, :n])


# speedup vs baseline: 1.2816x; 1.0653x over previous
import numpy as np

import jax
import jax.numpy as jnp
from jax.experimental import pallas as pl
from jax.experimental.pallas import tpu as pltpu

_TB = 512

_XSEQ = list(range(0, 32, 2)) + list(range(1, 32, 2))
_TAPS = {0: [(0, 15), (16, 31)],
         1: [(16, 31), (1, 16)],
         2: [(1, 16), (17, 32)]}


def _fused_kernel(x_ref,
                  m1_ref,
                  b1_ref,
                  m2_ref,
                  b2_ref,
                  fw1_ref, fb1_ref,
                  fw2_ref, fb2_ref,
                  fw3_ref, fb3_ref,
                  o_ref):
    f32 = jnp.float32
    vt3 = jnp.transpose(x_ref[...]).reshape(32, 32, _TB)

    xp = jnp.concatenate([vt3[:, c, :] for c in _XSEQ], axis=1)

    a1s = jnp.concatenate(
        [jnp.concatenate([xp[:, lo * _TB:hi * _TB] for lo, hi in _TAPS[j]],
                         axis=1)
         for j in range(3)], axis=0)
    c1 = jnp.dot(m1_ref[...], a1s, preferred_element_type=f32)

    px = jnp.maximum(c1[:, 0:15 * _TB], c1[:, 15 * _TB:30 * _TB])
    a2 = jnp.maximum(jnp.maximum(px[0:96, :], px[96:192, :])
                     + b1_ref[...], 0.0)

    c2 = (jnp.dot(m2_ref[0, :, :], a2[:, 0:13 * _TB],
                  preferred_element_type=f32)
          + jnp.dot(m2_ref[1, :, :], a2[:, _TB:14 * _TB],
                    preferred_element_type=f32)
          + jnp.dot(m2_ref[2, :, :], a2[:, 2 * _TB:15 * _TB],
                    preferred_element_type=f32))

    p2y = jnp.maximum(c2[0:96, :], c2[96:192, :])
    hf = jnp.concatenate(
        [jnp.maximum(p2y[:, 2 * k * _TB:(2 * k + 1) * _TB],
                     p2y[:, (2 * k + 1) * _TB:(2 * k + 2) * _TB])
         for k in range(6)], axis=0)
    hf = jnp.maximum(hf + b2_ref[...], 0.0)

    h3 = jnp.maximum(jnp.dot(fw1_ref[...], hf,
                             preferred_element_type=f32) + fb1_ref[...], 0.0)
    h4 = jnp.maximum(jnp.dot(fw2_ref[...], h3,
                             preferred_element_type=f32) + fb2_ref[...], 0.0)
    z = jnp.dot(fw3_ref[...], h4,
                preferred_element_type=f32) + fb3_ref[...]
    o_ref[...] = 1.0 / (1.0 + jnp.exp(-z))


def _band_tensor(n_out, n_in):
    b = np.zeros((2, 3, n_out, n_in), np.float32)
    for s in range(2):
        for i in range(3):
            b[s, i, np.arange(n_out), 2 * np.arange(n_out) + s + i] = 1.0
    return b


_B1 = _band_tensor(15, 32)
_B2 = _band_tensor(6, 15)


def _build_conv1_matrix(w1):
    w = w1[:, 0].astype(jnp.float32)
    m = jnp.einsum("cij,siyu->sycju", w, jnp.asarray(_B1))
    return jnp.pad(m.reshape(2, 90, 96), ((0, 0), (0, 6), (0, 0))).reshape(192, 96)


def _build_conv2_matrix(w2):
    w = w2.astype(jnp.float32)
    m = jnp.einsum("ocij,siyk->jsyokc", w, jnp.asarray(_B2))
    return jnp.pad(m.reshape(3, 192, 90), ((0, 0), (0, 0), (0, 6)))


def kernel(conv1_w, conv1_b, conv2_w, conv2_b, fc1_w, fc1_b,
           fc2_w, fc2_b, fc3_w, fc3_b, x_nchw):
    f32 = jnp.float32
    n = x_nchw.shape[0]
    n_pad = ((n + _TB - 1) // _TB) * _TB
    t = n_pad // _TB

    x = jnp.asarray(x_nchw, f32).reshape(n, 32 * 32)
    xa = jnp.pad(x, ((0, n_pad - n), (0, 0)))

    m1 = _build_conv1_matrix(conv1_w)
    b1c = jnp.pad(jnp.tile(conv1_b.astype(f32), (15,)), (0, 6)).reshape(96, 1)
    m2 = _build_conv2_matrix(conv2_w)
    b2c = jnp.tile(conv2_b.astype(f32), (36,)).reshape(576, 1)
    fw1 = fc1_w.reshape(128, 16, 6, 6).transpose(0, 3, 2, 1).reshape(128, 576)
    fw1 = fw1.astype(f32)
    fb1 = fc1_b.reshape(128, 1).astype(f32)
    fw2 = fc2_w.astype(f32)
    fb2 = fc2_b.reshape(64, 1).astype(f32)
    fw3 = fc3_w.astype(f32)
    fb3 = fc3_b.reshape(1, 1).astype(f32)

    out = pl.pallas_call(
        _fused_kernel,
        out_shape=jax.ShapeDtypeStruct((1, n_pad), f32),
        grid=(t,),
        in_specs=[
            pl.BlockSpec((_TB, 1024), lambda i: (i, 0)),
            pl.BlockSpec((192, 96), lambda i: (0, 0)),
            pl.BlockSpec((96, 1), lambda i: (0, 0)),
            pl.BlockSpec((3, 192, 96), lambda i: (0, 0, 0)),
            pl.BlockSpec((576, 1), lambda i: (0, 0)),
            pl.BlockSpec((128, 576), lambda i: (0, 0)),
            pl.BlockSpec((128, 1), lambda i: (0, 0)),
            pl.BlockSpec((64, 128), lambda i: (0, 0)),
            pl.BlockSpec((64, 1), lambda i: (0, 0)),
            pl.BlockSpec((1, 64), lambda i: (0, 0)),
            pl.BlockSpec((1, 1), lambda i: (0, 0)),
        ],
        out_specs=pl.BlockSpec((1, _TB), lambda i: (0, i)),
        compiler_params=pltpu.CompilerParams(
            dimension_semantics=("parallel",)),
    )(xa, m1, b1c, m2, b2c, fw1, fb1, fw2, fb2, fw3, fb3)

    return jnp.transpose(out[:, :n])


# speedup vs baseline: 1.3246x; 1.0336x over previous
import numpy as np

import jax
import jax.numpy as jnp
from jax.experimental import pallas as pl
from jax.experimental.pallas import tpu as pltpu

_TB = 1024

_XSEQ = list(range(0, 32, 2)) + list(range(1, 32, 2))
_TAPS = {0: [(0, 15), (16, 31)],
         1: [(16, 31), (1, 16)],
         2: [(1, 16), (17, 32)]}


def _fused_kernel(x_ref,
                  m1_ref,
                  b1_ref,
                  m2_ref,
                  b2_ref,
                  fw1_ref, fb1_ref,
                  fw2_ref, fb2_ref,
                  fw3_ref, fb3_ref,
                  o_ref):
    f32 = jnp.float32
    vt3 = jnp.transpose(x_ref[...]).reshape(32, 32, _TB)

    xp = jnp.concatenate([vt3[:, c, :] for c in _XSEQ], axis=1)

    a1s = jnp.concatenate(
        [jnp.concatenate([xp[:, lo * _TB:hi * _TB] for lo, hi in _TAPS[j]],
                         axis=1)
         for j in range(3)], axis=0)
    c1 = jnp.dot(m1_ref[...], a1s, preferred_element_type=f32)

    px = jnp.maximum(c1[:, 0:15 * _TB], c1[:, 15 * _TB:30 * _TB])
    a2 = jnp.maximum(jnp.maximum(px[0:96, :], px[96:192, :])
                     + b1_ref[...], 0.0)

    c2 = (jnp.dot(m2_ref[0, :, :], a2[:, 0:13 * _TB],
                  preferred_element_type=f32)
          + jnp.dot(m2_ref[1, :, :], a2[:, _TB:14 * _TB],
                    preferred_element_type=f32)
          + jnp.dot(m2_ref[2, :, :], a2[:, 2 * _TB:15 * _TB],
                    preferred_element_type=f32))

    p2y = jnp.maximum(c2[0:96, :], c2[96:192, :])
    hf = jnp.concatenate(
        [jnp.maximum(p2y[:, 2 * k * _TB:(2 * k + 1) * _TB],
                     p2y[:, (2 * k + 1) * _TB:(2 * k + 2) * _TB])
         for k in range(6)], axis=0)
    hf = jnp.maximum(hf + b2_ref[...], 0.0)

    h3 = jnp.maximum(jnp.dot(fw1_ref[...], hf,
                             preferred_element_type=f32) + fb1_ref[...], 0.0)
    h4 = jnp.maximum(jnp.dot(fw2_ref[...], h3,
                             preferred_element_type=f32) + fb2_ref[...], 0.0)
    z = jnp.dot(fw3_ref[...], h4,
                preferred_element_type=f32) + fb3_ref[...]
    o_ref[...] = 1.0 / (1.0 + jnp.exp(-z))


def _band_tensor(n_out, n_in):
    b = np.zeros((2, 3, n_out, n_in), np.float32)
    for s in range(2):
        for i in range(3):
            b[s, i, np.arange(n_out), 2 * np.arange(n_out) + s + i] = 1.0
    return b


_B1 = _band_tensor(15, 32)
_B2 = _band_tensor(6, 15)


def _build_conv1_matrix(w1):
    w = w1[:, 0].astype(jnp.float32)
    m = jnp.einsum("cij,siyu->sycju", w, jnp.asarray(_B1))
    return jnp.pad(m.reshape(2, 90, 96), ((0, 0), (0, 6), (0, 0))).reshape(192, 96)


def _build_conv2_matrix(w2):
    w = w2.astype(jnp.float32)
    m = jnp.einsum("ocij,siyk->jsyokc", w, jnp.asarray(_B2))
    return jnp.pad(m.reshape(3, 192, 90), ((0, 0), (0, 0), (0, 6)))


def kernel(conv1_w, conv1_b, conv2_w, conv2_b, fc1_w, fc1_b,
           fc2_w, fc2_b, fc3_w, fc3_b, x_nchw):
    f32 = jnp.float32
    n = x_nchw.shape[0]
    n_pad = ((n + _TB - 1) // _TB) * _TB
    t = n_pad // _TB

    x = jnp.asarray(x_nchw, f32).reshape(n, 32 * 32)
    xa = jnp.pad(x, ((0, n_pad - n), (0, 0)))

    m1 = _build_conv1_matrix(conv1_w)
    b1c = jnp.pad(jnp.tile(conv1_b.astype(f32), (15,)), (0, 6)).reshape(96, 1)
    m2 = _build_conv2_matrix(conv2_w)
    b2c = jnp.tile(conv2_b.astype(f32), (36,)).reshape(576, 1)
    fw1 = fc1_w.reshape(128, 16, 6, 6).transpose(0, 3, 2, 1).reshape(128, 576)
    fw1 = fw1.astype(f32)
    fb1 = fc1_b.reshape(128, 1).astype(f32)
    fw2 = fc2_w.astype(f32)
    fb2 = fc2_b.reshape(64, 1).astype(f32)
    fw3 = fc3_w.astype(f32)
    fb3 = fc3_b.reshape(1, 1).astype(f32)

    out = pl.pallas_call(
        _fused_kernel,
        out_shape=jax.ShapeDtypeStruct((1, n_pad), f32),
        grid=(t,),
        in_specs=[
            pl.BlockSpec((_TB, 1024), lambda i: (i, 0)),
            pl.BlockSpec((192, 96), lambda i: (0, 0)),
            pl.BlockSpec((96, 1), lambda i: (0, 0)),
            pl.BlockSpec((3, 192, 96), lambda i: (0, 0, 0)),
            pl.BlockSpec((576, 1), lambda i: (0, 0)),
            pl.BlockSpec((128, 576), lambda i: (0, 0)),
            pl.BlockSpec((128, 1), lambda i: (0, 0)),
            pl.BlockSpec((64, 128), lambda i: (0, 0)),
            pl.BlockSpec((64, 1), lambda i: (0, 0)),
            pl.BlockSpec((1, 64), lambda i: (0, 0)),
            pl.BlockSpec((1, 1), lambda i: (0, 0)),
        ],
        out_specs=pl.BlockSpec((1, _TB), lambda i: (0, i)),
        compiler_params=pltpu.CompilerParams(
            dimension_semantics=("parallel",)),
    )(xa, m1, b1c, m2, b2c, fw1, fb1, fw2, fb2, fw3, fb3)

    return jnp.transpose(out[:, :n])
